# Initial kernel scaffold; baseline (speedup 1.0000x reference)
import jax
import jax.numpy as jnp
from jax.experimental import pallas as pl


def kernel(data, x):
    raise NotImplementedError("write your pallas kernel here")


# speedup vs baseline: 1.7016x; 1.7016x over previous
import functools

import jax
import jax.numpy as jnp
from jax import lax
from jax.experimental import pallas as pl
from jax.experimental.pallas import tpu as pltpu
from jax.experimental.pallas import tpu_sc as plsc

_EPS = 1e-06

_N = 4194304
_C = 1024
_R = _N // _C
_BR = 512
_SC_C = 128
_SC_R = _N // _SC_C


def _interp_body(xs_ref, ds_ref, dsp_ref, val_ref):
    b = pl.program_id(0)
    row = lax.broadcasted_iota(jnp.int32, (_BR, _C), 0) + b * _BR
    col = lax.broadcasted_iota(jnp.int32, (_BR, _C), 1)
    i = row * _C + col
    first2 = i <= 1

    ds = ds_ref[...]
    dsp = dsp_ref[...]
    x1 = jnp.where(first2, ds_ref[0, 0], ds)
    x2 = jnp.where(first2, dsp_ref[0, 0], dsp)

    p = jnp.where(first2, 0, i)
    pm1 = jnp.where(p == 0, _N - 1, p - 1)

    rmax = jnp.float32(_N - 1)
    lo = jnp.float32(-1.0 + _EPS)
    hi = jnp.float32(1.0 - _EPS)

    def ramp(idx):
        z = (idx.astype(jnp.float32) / rmax - 0.5) * 2.0
        return lax.erf_inv(jnp.clip(z, lo, hi))

    y1 = ramp(p)
    y2 = ramp(pm1)

    rel = (xs_ref[...] - x2) / (x1 - x2)
    val_ref[...] = (1.0 - rel) * y2 + rel * y1


def _tc_interp(xs2, ds2, dsp2):
    return pl.pallas_call(
        _interp_body,
        grid=(_R // _BR,),
        in_specs=[pl.BlockSpec((_BR, _C), lambda b: (b, 0))] * 3,
        out_specs=pl.BlockSpec((_BR, _C), lambda b: (b, 0)),
        out_shape=jax.ShapeDtypeStruct((_R, _C), jnp.float32),
    )(xs2, ds2, dsp2)


def _sc_scatter(val2, idx2):
    info = plsc.get_sparse_core_info()
    nw = info.num_cores * info.num_subcores
    rows_per_w = _SC_R // nw
    chunk_rows = 16
    n_chunks = rows_per_w // chunk_rows

    mesh = plsc.VectorSubcoreMesh(core_axis_name="c", subcore_axis_name="s")

    @functools.partial(
        pl.kernel,
        out_type=jax.ShapeDtypeStruct((_N,), jnp.float32),
        mesh=mesh,
        scratch_types=[
            pltpu.VMEM((chunk_rows, _SC_C), jnp.float32),
            pltpu.VMEM((chunk_rows, _SC_C), jnp.int32),
            pltpu.SemaphoreType.DMA,
        ],
    )
    def scat(val_hbm, idx_hbm, out_hbm, val_v, idx_v, sem):
        wid = lax.axis_index("s") * info.num_cores + lax.axis_index("c")
        base_row = wid * rows_per_w

        def body(ci, carry):
            row0 = base_row + ci * chunk_rows
            pltpu.sync_copy(idx_hbm.at[pl.ds(row0, chunk_rows)], idx_v)
            pltpu.sync_copy(val_hbm.at[pl.ds(row0, chunk_rows)], val_v)
            cps = [
                pltpu.async_copy(val_v.at[j], out_hbm.at[idx_v.at[j]], sem)
                for j in range(chunk_rows)
            ]
            for cp in cps:
                cp.wait()
            return carry

        lax.fori_loop(0, n_chunks, body, 0)

    return scat(val2, idx2)


def kernel(data, x):
    n = data.shape[0]
    ds = lax.sort(data)
    xs, sx = lax.sort(
        (x, lax.iota(jnp.int32, n)), num_keys=1, is_stable=True
    )
    dsp = jnp.roll(ds, 1)
    val = _tc_interp(
        xs.reshape(_R, _C), ds.reshape(_R, _C), dsp.reshape(_R, _C)
    )
    return _sc_scatter(val.reshape(_SC_R, _SC_C), sx.reshape(_SC_R, _SC_C))


# speedup vs baseline: 1.7016x; 1.0000x over previous
import functools

import jax
import jax.numpy as jnp
from jax import lax
from jax.experimental import pallas as pl
from jax.experimental.pallas import tpu as pltpu
from jax.experimental.pallas import tpu_sc as plsc

_EPS = 1e-06

_N = 4194304
_C = 1024
_R = _N // _C
_BR = 512
_SC_C = 128
_SC_R = _N // _SC_C


def _interp_body(xs_ref, ds_ref, dsp_ref, val_ref):
    b = pl.program_id(0)
    row = lax.broadcasted_iota(jnp.int32, (_BR, _C), 0) + b * _BR
    col = lax.broadcasted_iota(jnp.int32, (_BR, _C), 1)
    i = row * _C + col
    first2 = i <= 1

    ds = ds_ref[...]
    dsp = dsp_ref[...]
    x1 = jnp.where(first2, ds_ref[0, 0], ds)
    x2 = jnp.where(first2, dsp_ref[0, 0], dsp)

    p = jnp.where(first2, 0, i)
    pm1 = jnp.where(p == 0, _N - 1, p - 1)

    rmax = jnp.float32(_N - 1)
    lo = jnp.float32(-1.0 + _EPS)
    hi = jnp.float32(1.0 - _EPS)

    def ramp(idx):
        z = (idx.astype(jnp.float32) / rmax - 0.5) * 2.0
        return lax.erf_inv(jnp.clip(z, lo, hi))

    y1 = ramp(p)
    y2 = ramp(pm1)

    rel = (xs_ref[...] - x2) / (x1 - x2)
    val_ref[...] = (1.0 - rel) * y2 + rel * y1


def _tc_interp(xs2, ds2, dsp2):
    return pl.pallas_call(
        _interp_body,
        grid=(_R // _BR,),
        in_specs=[pl.BlockSpec((_BR, _C), lambda b: (b, 0))] * 3,
        out_specs=pl.BlockSpec((_BR, _C), lambda b: (b, 0)),
        out_shape=jax.ShapeDtypeStruct((_R, _C), jnp.float32),
    )(xs2, ds2, dsp2)


def _sc_scatter(val2, idx2):
    info = plsc.get_sparse_core_info()
    nw = info.num_cores * info.num_subcores
    rows_per_w = _SC_R // nw
    chunk_rows = 64
    n_chunks = rows_per_w // chunk_rows

    mesh = plsc.VectorSubcoreMesh(core_axis_name="c", subcore_axis_name="s")

    @functools.partial(
        pl.kernel,
        out_type=jax.ShapeDtypeStruct((_N,), jnp.float32),
        mesh=mesh,
        scratch_types=[
            pltpu.VMEM((chunk_rows, _SC_C), jnp.float32),
            pltpu.VMEM((chunk_rows, _SC_C), jnp.int32),
            pltpu.SemaphoreType.DMA,
        ],
    )
    def scat(val_hbm, idx_hbm, out_hbm, val_v, idx_v, sem):
        wid = lax.axis_index("s") * info.num_cores + lax.axis_index("c")
        base_row = wid * rows_per_w

        def body(ci, carry):
            row0 = base_row + ci * chunk_rows
            pltpu.sync_copy(idx_hbm.at[pl.ds(row0, chunk_rows)], idx_v)
            pltpu.sync_copy(val_hbm.at[pl.ds(row0, chunk_rows)], val_v)
            cps = [
                pltpu.async_copy(val_v.at[j], out_hbm.at[idx_v.at[j]], sem)
                for j in range(chunk_rows)
            ]
            for cp in cps:
                cp.wait()
            return carry

        lax.fori_loop(0, n_chunks, body, 0)

    return scat(val2, idx2)


def kernel(data, x):
    n = data.shape[0]
    ds = lax.sort(data)
    xs, sx = lax.sort(
        (x, lax.iota(jnp.int32, n)), num_keys=1, is_stable=True
    )
    dsp = jnp.roll(ds, 1)
    val = _tc_interp(
        xs.reshape(_R, _C), ds.reshape(_R, _C), dsp.reshape(_R, _C)
    )
    return _sc_scatter(val.reshape(_SC_R, _SC_C), sx.reshape(_SC_R, _SC_C))


# speedup vs baseline: 1.7020x; 1.0002x over previous
import functools

import jax
import jax.numpy as jnp
from jax import lax
from jax.experimental import pallas as pl
from jax.experimental.pallas import tpu as pltpu
from jax.experimental.pallas import tpu_sc as plsc

_EPS = 1e-06

_N = 4194304
_C = 1024
_R = _N // _C
_BR = 512
_SC_C = 128
_SC_R = _N // _SC_C


def _interp_body(xs_ref, ds_ref, dsp_ref, val_ref):
    b = pl.program_id(0)
    row = lax.broadcasted_iota(jnp.int32, (_BR, _C), 0) + b * _BR
    col = lax.broadcasted_iota(jnp.int32, (_BR, _C), 1)
    i = row * _C + col
    first2 = i <= 1

    ds = ds_ref[...]
    dsp = dsp_ref[...]
    x1 = jnp.where(first2, ds_ref[0, 0], ds)
    x2 = jnp.where(first2, dsp_ref[0, 0], dsp)

    p = jnp.where(first2, 0, i)
    pm1 = jnp.where(p == 0, _N - 1, p - 1)

    rmax = jnp.float32(_N - 1)
    lo = jnp.float32(-1.0 + _EPS)
    hi = jnp.float32(1.0 - _EPS)

    def ramp(idx):
        z = (idx.astype(jnp.float32) / rmax - 0.5) * 2.0
        return lax.erf_inv(jnp.clip(z, lo, hi))

    y1 = ramp(p)
    y2 = ramp(pm1)

    rel = (xs_ref[...] - x2) / (x1 - x2)
    val_ref[...] = (1.0 - rel) * y2 + rel * y1


def _tc_interp(xs2, ds2, dsp2):
    return pl.pallas_call(
        _interp_body,
        grid=(_R // _BR,),
        in_specs=[pl.BlockSpec((_BR, _C), lambda b: (b, 0))] * 3,
        out_specs=pl.BlockSpec((_BR, _C), lambda b: (b, 0)),
        out_shape=jax.ShapeDtypeStruct((_R, _C), jnp.float32),
    )(xs2, ds2, dsp2)


def _sc_scatter(val2, idx2):
    info = plsc.get_sparse_core_info()
    nw = info.num_cores * info.num_subcores
    rows_per_w = _SC_R // nw
    chunk_rows = 64
    n_chunks = rows_per_w // chunk_rows

    mesh = plsc.VectorSubcoreMesh(core_axis_name="c", subcore_axis_name="s")

    @functools.partial(
        pl.kernel,
        out_type=jax.ShapeDtypeStruct((_N,), jnp.float32),
        mesh=mesh,
        scratch_types=[
            pltpu.VMEM((chunk_rows, _SC_C), jnp.float32),
            pltpu.VMEM((chunk_rows, _SC_C), jnp.int32),
            pltpu.VMEM((chunk_rows, _SC_C), jnp.float32),
            pltpu.VMEM((chunk_rows, _SC_C), jnp.int32),
            pltpu.SemaphoreType.DMA,
            pltpu.SemaphoreType.DMA,
        ],
    )
    def scat(val_hbm, idx_hbm, out_hbm, val_v0, idx_v0, val_v1, idx_v1,
             sem_ld, sem_sc):
        wid = lax.axis_index("s") * info.num_cores + lax.axis_index("c")
        base_row = wid * rows_per_w
        bufs = ((val_v0, idx_v0), (val_v1, idx_v1))

        def load(ci, vb, ib):
            row0 = base_row + ci * chunk_rows
            return [
                pltpu.async_copy(idx_hbm.at[pl.ds(row0, chunk_rows)], ib,
                                 sem_ld),
                pltpu.async_copy(val_hbm.at[pl.ds(row0, chunk_rows)], vb,
                                 sem_ld),
            ]

        lds = load(0, *bufs[0])
        for ci in range(n_chunks):
            vb, ib = bufs[ci % 2]
            if ci + 1 < n_chunks:
                lds_next = load(ci + 1, *bufs[(ci + 1) % 2])
            for cp in lds:
                cp.wait()
            scs = [
                pltpu.async_copy(vb.at[j], out_hbm.at[ib.at[j]], sem_sc)
                for j in range(chunk_rows)
            ]
            if ci + 1 < n_chunks:
                lds = lds_next
            for cp in scs:
                cp.wait()

    return scat(val2, idx2)


def kernel(data, x):
    n = data.shape[0]
    ds = lax.sort(data)
    xs, sx = lax.sort(
        (x, lax.iota(jnp.int32, n)), num_keys=1, is_stable=True
    )
    dsp = jnp.roll(ds, 1)
    val = _tc_interp(
        xs.reshape(_R, _C), ds.reshape(_R, _C), dsp.reshape(_R, _C)
    )
    return _sc_scatter(val.reshape(_SC_R, _SC_C), sx.reshape(_SC_R, _SC_C))
